# baseline (speedup 1.0000x reference)
import functools

import jax
import jax.numpy as jnp
from jax import lax
from jax.experimental import pallas as pl
from jax.experimental.pallas import tpu as pltpu
from jax.experimental.pallas import tpu_sc as plsc

L = 16
NB = 4
CH = 4
NB_TC = 3


def _tc_body(mid_ref, x_ref, pe_ref, table_ref, out_ref):
    mid = mid_ref[0]
    row = table_ref[pl.ds(mid, 1), :]
    out_ref[...] = x_ref[...] + pe_ref[...] + row[None, :, :]


def _tc_part(x, temporal_pe, modality_table, mid, nb):
    B, T, D = x.shape
    TB = 2048
    nt = T // TB

    grid_spec = pltpu.PrefetchScalarGridSpec(
        num_scalar_prefetch=1,
        grid=(nt, nb),
        in_specs=[
            pl.BlockSpec((1, TB, D), lambda t, b, mid: (b, t, 0)),
            pl.BlockSpec((1, TB, D), lambda t, b, mid: (0, t, 0)),
            pl.BlockSpec(modality_table.shape, lambda t, b, mid: (0, 0)),
        ],
        out_specs=pl.BlockSpec((1, TB, D), lambda t, b, mid: (b, t, 0)),
    )

    return pl.pallas_call(
        _tc_body,
        grid_spec=grid_spec,
        out_shape=jax.ShapeDtypeStruct((nb, T, D), x.dtype),
        compiler_params=pltpu.CompilerParams(
            dimension_semantics=("arbitrary", "arbitrary"),
        ),
    )(mid, x, temporal_pe, modality_table)


def _sc_body(B, T, D, b_lo, x_hbm, pe_hbm, table_hbm, mid_hbm, out_hbm,
             idx_v, me_v, pe_v, x_v, sem_in, sem_out, gsem):
    nbat = B - b_lo
    c = lax.axis_index("c")
    s = lax.axis_index("s")
    nc = lax.axis_size("c")
    ns = lax.axis_size("s")
    nw = nc * ns
    wid = s * nc + c

    pltpu.sync_copy(mid_hbm, idx_v)
    pltpu.async_copy(table_hbm.at[idx_v], me_v, gsem).wait()

    t_per_w = T // nw
    n_ch = t_per_w // CH
    base = wid * t_per_w

    def in_copies(k, sl):
        t0 = base + k * CH
        return [
            pltpu.make_async_copy(
                pe_hbm.at[pl.ds(t0, CH)], pe_v.at[sl], sem_in.at[sl]),
            pltpu.make_async_copy(
                x_hbm.at[pl.ds(b_lo, nbat), pl.ds(t0, CH), :], x_v.at[sl],
                sem_in.at[sl]),
        ]

    def out_copies(k, sl):
        t0 = base + k * CH
        return [pltpu.make_async_copy(
            x_v.at[sl, b], out_hbm.at[b, pl.ds(t0, CH), :], sem_out.at[sl])
            for b in range(nbat)]

    def compute(sl):
        pe_sl = pe_v.at[sl]
        x_sl = x_v.at[sl]

        @plsc.parallel_loop(0, D // L, unroll=8)
        def _(j):
            slc = pl.ds(j * L, L)
            mv = me_v[0, slc]
            for r in range(CH):
                pv = pe_sl[r, slc] + mv
                for b in range(nbat):
                    plsc.addupdate(x_sl.at[b, r, slc], pv)

    def step(k, sl, wait_prev_out, issue_next):
        for cp in in_copies(k, sl):
            cp.wait()
        compute(sl)
        for cp in out_copies(k, sl):
            cp.start()
        nsl = (sl + NB - 1) % NB
        if wait_prev_out:
            for cp in out_copies(k - 1, nsl):
                cp.wait()
        if issue_next:
            for cp in in_copies(k + NB - 1, nsl):
                cp.start()

    for p in range(NB - 1):
        for cp in in_copies(p, p):
            cp.start()

    step(0, 0, False, True)
    for k0 in range(1, NB):
        step(k0, k0, True, True)

    def steady(g, carry):
        for sl in range(NB):
            step(g * NB + sl, sl, True, True)
        return carry

    lax.fori_loop(1, n_ch // NB - 1, steady, 0)

    kL = n_ch - NB
    step(kL, 0, True, True)
    for sl in range(1, NB):
        step(kL + sl, sl, False, False)

    for kk in range(n_ch - NB, n_ch):
        for cp in out_copies(kk, kk % NB):
            cp.wait()


def _sc_part(x, pe2, modality_table, mid, b_lo):
    B, T, D = x.shape
    nbat = B - b_lo
    mesh = plsc.VectorSubcoreMesh(core_axis_name="c", subcore_axis_name="s")
    body = functools.partial(_sc_body, B, T, D, b_lo)
    k = pl.kernel(
        body,
        mesh=mesh,
        out_type=jax.ShapeDtypeStruct((nbat, T, D), x.dtype),
        scratch_types=[
            pltpu.VMEM((1,), jnp.int32),
            pltpu.VMEM((1, D), jnp.float32),
            pltpu.VMEM((NB, CH, D), jnp.float32),
            pltpu.VMEM((NB, nbat, CH, D), jnp.float32),
            pltpu.SemaphoreType.DMA((NB,)),
            pltpu.SemaphoreType.DMA((NB,)),
            pltpu.SemaphoreType.DMA,
        ],
    )
    return k(x, pe2, modality_table, mid)


@jax.jit
def kernel(x, temporal_pe, modality_table, modality_id):
    B, T, D = x.shape
    pe2 = temporal_pe.reshape(temporal_pe.shape[1], D)
    mid = jnp.asarray(modality_id, jnp.int32).reshape(1)

    out_tc = _tc_part(x, temporal_pe, modality_table, mid, NB_TC)
    out_sc = _sc_part(x, pe2, modality_table, mid, NB_TC)
    return jnp.concatenate([out_tc, out_sc], axis=0)


# speedup vs baseline: 2.4105x; 2.4105x over previous
import jax
import jax.numpy as jnp
from jax.experimental import pallas as pl
from jax.experimental.pallas import tpu as pltpu


def _body(mid_ref, x_ref, pe_ref, table_ref, out_ref):
    mid = mid_ref[0]
    row = table_ref[pl.ds(mid, 1), :]
    out_ref[...] = x_ref[...] + pe_ref[...] + row[None, :, :]


@jax.jit
def kernel(x, temporal_pe, modality_table, modality_id):
    B, T, D = x.shape
    TB = 2048
    nt = T // TB
    mid = jnp.asarray(modality_id, jnp.int32).reshape(1)

    grid_spec = pltpu.PrefetchScalarGridSpec(
        num_scalar_prefetch=1,
        grid=(nt, B),
        in_specs=[
            pl.BlockSpec((1, TB, D), lambda t, b, mid: (b, t, 0)),
            pl.BlockSpec((1, TB, D), lambda t, b, mid: (0, t, 0)),
            pl.BlockSpec(modality_table.shape, lambda t, b, mid: (0, 0)),
        ],
        out_specs=pl.BlockSpec((1, TB, D), lambda t, b, mid: (b, t, 0)),
    )

    return pl.pallas_call(
        _body,
        grid_spec=grid_spec,
        out_shape=jax.ShapeDtypeStruct((B, T, D), x.dtype),
        compiler_params=pltpu.CompilerParams(
            dimension_semantics=("arbitrary", "arbitrary"),
        ),
    )(mid, x, temporal_pe, modality_table)


# speedup vs baseline: 2.4183x; 1.0032x over previous
import jax
import jax.numpy as jnp
from jax.experimental import pallas as pl
from jax.experimental.pallas import tpu as pltpu


def _body(mid_ref, x_ref, pe_ref, table_ref, out_ref):
    mid = mid_ref[0]
    row = table_ref[pl.ds(mid, 1), :]
    out_ref[...] = x_ref[...] + pe_ref[...] + row[None, :, :]


@jax.jit
def kernel(x, temporal_pe, modality_table, modality_id):
    B, T, D = x.shape
    TB = 2048
    nt = T // TB
    mid = jnp.asarray(modality_id, jnp.int32).reshape(1)

    grid_spec = pltpu.PrefetchScalarGridSpec(
        num_scalar_prefetch=1,
        grid=(nt, B),
        in_specs=[
            pl.BlockSpec((1, TB, D), lambda t, b, mid: (b, t, 0)),
            pl.BlockSpec((1, TB, D), lambda t, b, mid: (0, t, 0)),
            pl.BlockSpec(modality_table.shape, lambda t, b, mid: (0, 0)),
        ],
        out_specs=pl.BlockSpec((1, TB, D), lambda t, b, mid: (b, t, 0)),
    )

    return pl.pallas_call(
        _body,
        grid_spec=grid_spec,
        out_shape=jax.ShapeDtypeStruct((B, T, D), x.dtype),
        compiler_params=pltpu.CompilerParams(
            dimension_semantics=("parallel", "parallel"),
        ),
    )(mid, x, temporal_pe, modality_table)
